# Initial kernel scaffold; baseline (speedup 1.0000x reference)
import jax
import jax.numpy as jnp
from jax.experimental import pallas as pl


def kernel(x, edge_index, batch, W1, b1, W2, b2, Wfc, bfc):
    raise NotImplementedError("write your pallas kernel here")


# speedup vs baseline: 10.4163x; 10.4163x over previous
import functools

import jax
import jax.numpy as jnp
from jax import lax
from jax.experimental import pallas as pl
from jax.experimental.pallas import tpu as pltpu
from jax.experimental.pallas import tpu_sc as plsc

N = 10000
E = 320000
D = 128
G = 64

NW = 32
EPW = E // NW
CH = 80
NCH = EPW // CH
NH = N // 2
ACC_ROWS = NH + 8

_mesh = plsc.VectorSubcoreMesh(core_axis_name="c", subcore_axis_name="s")


@functools.partial(
    pl.kernel,
    out_type=jax.ShapeDtypeStruct((NW, N), jnp.float32),
    mesh=_mesh,
    scratch_types=[
        pltpu.VMEM((EPW // 16, 16), jnp.int32),
        pltpu.VMEM((N,), jnp.float32),
    ],
    compiler_params=pltpu.CompilerParams(needs_layout_passes=False),
)
def _deg_kernel(dst_hbm, out_hbm, idx_v, hist_v):
    sid = lax.axis_index("s")
    cid = lax.axis_index("c")
    wid = sid * 2 + cid

    z16 = jnp.zeros((16,), jnp.float32)

    @pl.loop(0, N // 16)
    def _zero(i):
        hist_v[pl.ds(i * 16, 16)] = z16

    pltpu.sync_copy(dst_hbm.at[wid], idx_v)
    ones16 = jnp.ones((16,), jnp.float32)

    @pl.loop(0, EPW // 16)
    def _hist(i):
        plsc.addupdate_scatter(hist_v, [idx_v[i]], ones16)

    pltpu.sync_copy(hist_v, out_hbm.at[wid])


@functools.partial(
    pl.kernel,
    out_type=jax.ShapeDtypeStruct((2, N, D), jnp.float32),
    mesh=_mesh,
    scratch_types=[
        pltpu.VMEM((NCH, CH), jnp.int32),
        pltpu.VMEM((NCH, CH), jnp.int32),
        pltpu.VMEM((8, CH), jnp.int32),
        pltpu.VMEM((CH, D), jnp.float32),
        pltpu.VMEM((200, D), jnp.float32),
        pltpu.VMEM_SHARED((ACC_ROWS, D), jnp.float32),
        pltpu.SemaphoreType.DMA,
    ],
)
def _msg_kernel(hn_hbm, src_hbm, dst_hbm, out_hbm,
                src_v, dst_v, idxb, rows_v, zbuf, acc_sh, sem):
    sid = lax.axis_index("s")
    cid = lax.axis_index("c")
    wid = sid * 2 + cid

    z16 = jnp.zeros((16,), jnp.float32)

    @pl.loop(0, 200)
    def _zrow(i):
        @pl.loop(0, D // 16)
        def _zcol(c):
            zbuf[i, pl.ds(c * 16, 16)] = z16

    pltpu.sync_copy(src_hbm.at[wid], src_v)
    pltpu.sync_copy(dst_hbm.at[wid], dst_v)

    for p in range(2):
        base = p * NH

        @pl.when(sid < 5)
        def _zacc():
            @pl.loop(0, 5)
            def _zk(k):
                pltpu.sync_copy(
                    zbuf, acc_sh.at[pl.ds(sid * 1000 + k * 200, 200), :])

        @pl.when(sid == 5)
        def _zdump():
            pltpu.sync_copy(zbuf.at[pl.ds(0, 8), :],
                            acc_sh.at[pl.ds(NH, 8), :])

        plsc.subcore_barrier()

        @pl.loop(0, NCH)
        def _chunk(j):
            cp = pltpu.async_copy(hn_hbm.at[src_v.at[j]], rows_v, sem)
            dst_row = dst_v.at[j]
            for k in range(CH // 16):
                v = dst_row[pl.ds(k * 16, 16)] - base
                inr = (v >= 0) & (v < NH)
                idxb[0, pl.ds(k * 16, 16)] = jnp.where(inr, v, NH)
            cp.wait()
            pltpu.sync_copy(rows_v, acc_sh.at[idxb.at[0]], add=True)

        plsc.subcore_barrier()

        @pl.when(sid < 5)
        def _writeout():
            pltpu.sync_copy(
                acc_sh.at[pl.ds(sid * 1000, 1000), :],
                out_hbm.at[cid, pl.ds(base + sid * 1000, 1000), :],
            )

        plsc.subcore_barrier()


def _tc_a_body(x_ref, w_ref, dg_ref, out_ref):
    deg = jnp.sum(dg_ref[:, :], axis=1) + 1.0
    dinv = lax.rsqrt(deg)
    h = jnp.dot(x_ref[:, :], w_ref[:, :], preferred_element_type=jnp.float32)
    out_ref[:, :] = h * dinv[:, None]


def _tc_a(x, w, dg, rows):
    grid = x.shape[0] // rows
    return pl.pallas_call(
        _tc_a_body,
        grid=(grid,),
        in_specs=[
            pl.BlockSpec((rows, D), lambda i: (i, 0)),
            pl.BlockSpec((D, D), lambda i: (0, 0)),
            pl.BlockSpec((rows, NW), lambda i: (i, 0)),
        ],
        out_specs=pl.BlockSpec((rows, D), lambda i: (i, 0)),
        out_shape=jax.ShapeDtypeStruct((x.shape[0], D), jnp.float32),
    )(x, w, dg)


def _tc_c_body(a0_ref, a1_ref, hn_ref, dg_ref, b_ref, w_ref, out_ref):
    deg = jnp.sum(dg_ref[:, :], axis=1) + 1.0
    dinv = lax.rsqrt(deg)
    t = (a0_ref[:, :] + a1_ref[:, :] + hn_ref[:, :]) * dinv[:, None]
    t = jnp.maximum(t + b_ref[:, :], 0.0)
    h = jnp.dot(t, w_ref[:, :], preferred_element_type=jnp.float32)
    out_ref[:, :] = h * dinv[:, None]


def _tc_c(a0, a1, hn, dg, b, w, rows):
    grid = hn.shape[0] // rows
    return pl.pallas_call(
        _tc_c_body,
        grid=(grid,),
        in_specs=[
            pl.BlockSpec((rows, D), lambda i: (i, 0)),
            pl.BlockSpec((rows, D), lambda i: (i, 0)),
            pl.BlockSpec((rows, D), lambda i: (i, 0)),
            pl.BlockSpec((rows, NW), lambda i: (i, 0)),
            pl.BlockSpec((1, D), lambda i: (0, 0)),
            pl.BlockSpec((D, D), lambda i: (0, 0)),
        ],
        out_specs=pl.BlockSpec((rows, D), lambda i: (i, 0)),
        out_shape=jax.ShapeDtypeStruct((hn.shape[0], D), jnp.float32),
    )(a0, a1, hn, dg, b, w)


def _tc_e_body(a0_ref, a1_ref, hn_ref, dg_ref, b_ref, batch_ref,
               wfc_ref, bfc_ref, out_ref, gsum, gcnt):
    i = pl.program_id(0)

    @pl.when(i == 0)
    def _init():
        gsum[:, :] = jnp.zeros_like(gsum)
        gcnt[:, :] = jnp.zeros_like(gcnt)

    deg = jnp.sum(dg_ref[:, :], axis=1) + 1.0
    dinv = lax.rsqrt(deg)
    t = (a0_ref[:, :] + a1_ref[:, :] + hn_ref[:, :]) * dinv[:, None]
    h3 = jnp.maximum(t + b_ref[:, :], 0.0)

    rows = h3.shape[0]
    gids = lax.broadcasted_iota(jnp.int32, (G, rows), 0)
    mask = (batch_ref[:, 0][None, :] == gids).astype(jnp.float32)
    gsum[:, :] += jnp.dot(mask, h3, preferred_element_type=jnp.float32)
    cnt = jnp.sum(mask, axis=1, keepdims=True)
    gcnt[:, :] += jnp.broadcast_to(cnt, (G, D))

    @pl.when(i == pl.num_programs(0) - 1)
    def _fin():
        g = gsum[:, :] / jnp.maximum(gcnt[:, :], 1.0)
        out_ref[:, :] = (
            jnp.dot(g, wfc_ref[:, :], preferred_element_type=jnp.float32)
            + bfc_ref[:, :])


def _tc_e(a0, a1, hn, dg, b, batch2d, wfc, bfc, rows):
    grid = hn.shape[0] // rows
    return pl.pallas_call(
        _tc_e_body,
        grid=(grid,),
        in_specs=[
            pl.BlockSpec((rows, D), lambda i: (i, 0)),
            pl.BlockSpec((rows, D), lambda i: (i, 0)),
            pl.BlockSpec((rows, D), lambda i: (i, 0)),
            pl.BlockSpec((rows, NW), lambda i: (i, 0)),
            pl.BlockSpec((1, D), lambda i: (0, 0)),
            pl.BlockSpec((rows, 1), lambda i: (i, 0)),
            pl.BlockSpec((D, D), lambda i: (0, 0)),
            pl.BlockSpec((1, D), lambda i: (0, 0)),
        ],
        out_specs=pl.BlockSpec((G, D), lambda i: (0, 0)),
        out_shape=jax.ShapeDtypeStruct((G, D), jnp.float32),
        scratch_shapes=[
            pltpu.VMEM((G, D), jnp.float32),
            pltpu.VMEM((G, D), jnp.float32),
        ],
    )(a0, a1, hn, dg, b, batch2d, wfc, bfc)


def kernel(x, edge_index, batch, W1, b1, W2, b2, Wfc, bfc):
    src = edge_index[0]
    dst = edge_index[1]
    src_r = src.reshape(NW, NCH, CH)
    dst_r = dst.reshape(NW, NCH, CH)
    dst_deg = dst.reshape(NW, EPW // 16, 16)

    deg_p = _deg_kernel(dst_deg)
    dg = deg_p.T

    rows = 1000
    hn1 = _tc_a(x, W1, dg, rows)
    acc1 = _msg_kernel(hn1, src_r, dst_r)
    hn2 = _tc_c(acc1[0], acc1[1], hn1, dg,
                b1.reshape(1, D), W2, rows)
    acc2 = _msg_kernel(hn2, src_r, dst_r)
    out = _tc_e(acc2[0], acc2[1], hn2, dg,
                b2.reshape(1, D), batch.reshape(N, 1),
                Wfc, bfc.reshape(1, D), rows)
    return out


# speedup vs baseline: 14.6691x; 1.4083x over previous
import functools

import jax
import jax.numpy as jnp
from jax import lax
from jax.experimental import pallas as pl
from jax.experimental.pallas import tpu as pltpu
from jax.experimental.pallas import tpu_sc as plsc

N = 10000
E = 320000
D = 128
G = 64

NW = 32
EPW = E // NW
CH = 80
NCH = EPW // CH
NH = N // 2
ACC_ROWS = NH + 8

_mesh = plsc.VectorSubcoreMesh(core_axis_name="c", subcore_axis_name="s")


@functools.partial(
    pl.kernel,
    out_type=jax.ShapeDtypeStruct((NW, N), jnp.float32),
    mesh=_mesh,
    scratch_types=[
        pltpu.VMEM((EPW // 16, 16), jnp.int32),
        pltpu.VMEM((N,), jnp.float32),
    ],
    compiler_params=pltpu.CompilerParams(needs_layout_passes=False),
)
def _deg_kernel(dst_hbm, out_hbm, idx_v, hist_v):
    sid = lax.axis_index("s")
    cid = lax.axis_index("c")
    wid = sid * 2 + cid

    z16 = jnp.zeros((16,), jnp.float32)

    @pl.loop(0, N // 16)
    def _zero(i):
        hist_v[pl.ds(i * 16, 16)] = z16

    pltpu.sync_copy(dst_hbm.at[wid], idx_v)
    ones16 = jnp.ones((16,), jnp.float32)

    @pl.loop(0, EPW // 16)
    def _hist(i):
        plsc.addupdate_scatter(hist_v, [idx_v[i]], ones16)

    pltpu.sync_copy(hist_v, out_hbm.at[wid])


@functools.partial(
    pl.kernel,
    out_type=jax.ShapeDtypeStruct((2, N, D), jnp.float32),
    mesh=_mesh,
    scratch_types=[
        pltpu.VMEM((NCH, CH), jnp.int32),
        pltpu.VMEM((NCH, CH), jnp.int32),
        pltpu.VMEM((8, CH), jnp.int32),
        pltpu.VMEM((CH, D), jnp.float32),
        pltpu.VMEM((CH, D), jnp.float32),
        pltpu.VMEM((200, D), jnp.float32),
        pltpu.VMEM_SHARED((ACC_ROWS, D), jnp.float32),
        pltpu.SemaphoreType.DMA,
        pltpu.SemaphoreType.DMA,
    ],
)
def _msg_kernel(hn_hbm, src_hbm, dst_hbm, out_hbm,
                src_v, dst_v, idxb, rows0, rows1, zbuf, acc_sh, sem0, sem1):
    sid = lax.axis_index("s")
    cid = lax.axis_index("c")
    wid = sid * 2 + cid

    z16 = jnp.zeros((16,), jnp.float32)

    @pl.loop(0, 200)
    def _zrow(i):
        @pl.loop(0, D // 16)
        def _zcol(c):
            zbuf[i, pl.ds(c * 16, 16)] = z16

    pltpu.sync_copy(src_hbm.at[wid], src_v)
    pltpu.sync_copy(dst_hbm.at[wid], dst_v)

    for p in range(2):
        base = p * NH

        @pl.when(sid < 5)
        def _zacc():
            @pl.loop(0, 5)
            def _zk(k):
                pltpu.sync_copy(
                    zbuf, acc_sh.at[pl.ds(sid * 1000 + k * 200, 200), :])

        @pl.when(sid == 5)
        def _zdump():
            pltpu.sync_copy(zbuf.at[pl.ds(0, 8), :],
                            acc_sh.at[pl.ds(NH, 8), :])

        plsc.subcore_barrier()

        def _start(j, buf, sem_):
            pltpu.async_copy(hn_hbm.at[src_v.at[j]], buf, sem_)

        def _finish(j, buf, sem_, row):
            dst_row = dst_v.at[j]
            for k in range(CH // 16):
                v = dst_row[pl.ds(k * 16, 16)] - base
                inr = (v >= 0) & (v < NH)
                idxb[row, pl.ds(k * 16, 16)] = jnp.where(inr, v, NH)
            pltpu.make_async_copy(hn_hbm.at[pl.ds(0, CH), :], buf, sem_).wait()
            pltpu.sync_copy(buf, acc_sh.at[idxb.at[row]], add=True)

        _start(0, rows0, sem0)

        @pl.loop(0, NCH - 1, step=2)
        def _chunk(j):
            _start(j + 1, rows1, sem1)
            _finish(j, rows0, sem0, 0)
            _start(j + 2, rows0, sem0)
            _finish(j + 1, rows1, sem1, 1)

        _finish(NCH - 1, rows0, sem0, 0)

        plsc.subcore_barrier()

        @pl.when(sid < 5)
        def _writeout():
            pltpu.sync_copy(
                acc_sh.at[pl.ds(sid * 1000, 1000), :],
                out_hbm.at[cid, pl.ds(base + sid * 1000, 1000), :],
            )

        plsc.subcore_barrier()


def _tc_a_body(x_ref, w_ref, dg_ref, out_ref):
    deg = jnp.sum(dg_ref[:, :], axis=1) + 1.0
    dinv = lax.rsqrt(deg)
    h = jnp.dot(x_ref[:, :], w_ref[:, :], preferred_element_type=jnp.float32)
    out_ref[:, :] = h * dinv[:, None]


def _tc_a(x, w, dg, rows):
    grid = x.shape[0] // rows
    return pl.pallas_call(
        _tc_a_body,
        grid=(grid,),
        in_specs=[
            pl.BlockSpec((rows, D), lambda i: (i, 0)),
            pl.BlockSpec((D, D), lambda i: (0, 0)),
            pl.BlockSpec((rows, NW), lambda i: (i, 0)),
        ],
        out_specs=pl.BlockSpec((rows, D), lambda i: (i, 0)),
        out_shape=jax.ShapeDtypeStruct((x.shape[0], D), jnp.float32),
    )(x, w, dg)


def _tc_c_body(a0_ref, a1_ref, hn_ref, dg_ref, b_ref, w_ref, out_ref):
    deg = jnp.sum(dg_ref[:, :], axis=1) + 1.0
    dinv = lax.rsqrt(deg)
    t = (a0_ref[:, :] + a1_ref[:, :] + hn_ref[:, :]) * dinv[:, None]
    t = jnp.maximum(t + b_ref[:, :], 0.0)
    h = jnp.dot(t, w_ref[:, :], preferred_element_type=jnp.float32)
    out_ref[:, :] = h * dinv[:, None]


def _tc_c(a0, a1, hn, dg, b, w, rows):
    grid = hn.shape[0] // rows
    return pl.pallas_call(
        _tc_c_body,
        grid=(grid,),
        in_specs=[
            pl.BlockSpec((rows, D), lambda i: (i, 0)),
            pl.BlockSpec((rows, D), lambda i: (i, 0)),
            pl.BlockSpec((rows, D), lambda i: (i, 0)),
            pl.BlockSpec((rows, NW), lambda i: (i, 0)),
            pl.BlockSpec((1, D), lambda i: (0, 0)),
            pl.BlockSpec((D, D), lambda i: (0, 0)),
        ],
        out_specs=pl.BlockSpec((rows, D), lambda i: (i, 0)),
        out_shape=jax.ShapeDtypeStruct((hn.shape[0], D), jnp.float32),
    )(a0, a1, hn, dg, b, w)


def _tc_e_body(a0_ref, a1_ref, hn_ref, dg_ref, b_ref, batch_ref,
               wfc_ref, bfc_ref, out_ref, gsum, gcnt):
    i = pl.program_id(0)

    @pl.when(i == 0)
    def _init():
        gsum[:, :] = jnp.zeros_like(gsum)
        gcnt[:, :] = jnp.zeros_like(gcnt)

    deg = jnp.sum(dg_ref[:, :], axis=1) + 1.0
    dinv = lax.rsqrt(deg)
    t = (a0_ref[:, :] + a1_ref[:, :] + hn_ref[:, :]) * dinv[:, None]
    h3 = jnp.maximum(t + b_ref[:, :], 0.0)

    rows = h3.shape[0]
    gids = lax.broadcasted_iota(jnp.int32, (G, rows), 0)
    mask = (batch_ref[:, 0][None, :] == gids).astype(jnp.float32)
    gsum[:, :] += jnp.dot(mask, h3, preferred_element_type=jnp.float32)
    cnt = jnp.sum(mask, axis=1, keepdims=True)
    gcnt[:, :] += jnp.broadcast_to(cnt, (G, D))

    @pl.when(i == pl.num_programs(0) - 1)
    def _fin():
        g = gsum[:, :] / jnp.maximum(gcnt[:, :], 1.0)
        out_ref[:, :] = (
            jnp.dot(g, wfc_ref[:, :], preferred_element_type=jnp.float32)
            + bfc_ref[:, :])


def _tc_e(a0, a1, hn, dg, b, batch2d, wfc, bfc, rows):
    grid = hn.shape[0] // rows
    return pl.pallas_call(
        _tc_e_body,
        grid=(grid,),
        in_specs=[
            pl.BlockSpec((rows, D), lambda i: (i, 0)),
            pl.BlockSpec((rows, D), lambda i: (i, 0)),
            pl.BlockSpec((rows, D), lambda i: (i, 0)),
            pl.BlockSpec((rows, NW), lambda i: (i, 0)),
            pl.BlockSpec((1, D), lambda i: (0, 0)),
            pl.BlockSpec((rows, 1), lambda i: (i, 0)),
            pl.BlockSpec((D, D), lambda i: (0, 0)),
            pl.BlockSpec((1, D), lambda i: (0, 0)),
        ],
        out_specs=pl.BlockSpec((G, D), lambda i: (0, 0)),
        out_shape=jax.ShapeDtypeStruct((G, D), jnp.float32),
        scratch_shapes=[
            pltpu.VMEM((G, D), jnp.float32),
            pltpu.VMEM((G, D), jnp.float32),
        ],
    )(a0, a1, hn, dg, b, batch2d, wfc, bfc)


def kernel(x, edge_index, batch, W1, b1, W2, b2, Wfc, bfc):
    src = edge_index[0]
    dst = edge_index[1]
    src_r = src.reshape(NW, NCH, CH)
    dst_r = dst.reshape(NW, NCH, CH)
    dst_deg = dst.reshape(NW, EPW // 16, 16)

    deg_p = _deg_kernel(dst_deg)
    dg = deg_p.T

    rows = 1000
    hn1 = _tc_a(x, W1, dg, rows)
    acc1 = _msg_kernel(hn1, src_r, dst_r)
    hn2 = _tc_c(acc1[0], acc1[1], hn1, dg,
                b1.reshape(1, D), W2, rows)
    acc2 = _msg_kernel(hn2, src_r, dst_r)
    out = _tc_e(acc2[0], acc2[1], hn2, dg,
                b2.reshape(1, D), batch.reshape(N, 1),
                Wfc, bfc.reshape(1, D), rows)
    return out


# speedup vs baseline: 16.1280x; 1.0995x over previous
import functools

import jax
import jax.numpy as jnp
from jax import lax
from jax.experimental import pallas as pl
from jax.experimental.pallas import tpu as pltpu
from jax.experimental.pallas import tpu_sc as plsc

N = 10000
E = 320000
D = 128
G = 64

NW = 32
EPW = E // NW
CH = 80
NCH = EPW // CH
NH = N // 2
ACC_ROWS = NH + 8
LW = EPW + 2 * CH
LCH = LW // CH

_mesh = plsc.VectorSubcoreMesh(core_axis_name="c", subcore_axis_name="s")


@functools.partial(
    pl.kernel,
    out_type=(
        jax.ShapeDtypeStruct((NW, N), jnp.float32),
        jax.ShapeDtypeStruct((NW, LW), jnp.int32),
        jax.ShapeDtypeStruct((NW, LW), jnp.int32),
        jax.ShapeDtypeStruct((NW, 2, 16), jnp.int32),
    ),
    mesh=_mesh,
    scratch_types=[
        pltpu.VMEM((125, 16), jnp.int32),
        pltpu.VMEM((125, 16), jnp.int32),
        pltpu.VMEM((N,), jnp.float32),
        pltpu.VMEM((LW,), jnp.int32),
        pltpu.VMEM((LW,), jnp.int32),
        pltpu.VMEM((2, 16), jnp.int32),
    ],
    compiler_params=pltpu.CompilerParams(needs_layout_passes=False),
)
def _deg_kernel(src_hbm, dst_hbm, deg_hbm, elo_hbm, ehi_hbm, cnt_hbm,
                src_v, dst_v, hist_v, elo_v, ehi_v, cnt_v):
    sid = lax.axis_index("s")
    cid = lax.axis_index("c")
    wid = sid * 2 + cid

    z16 = jnp.zeros((16,), jnp.float32)

    @pl.loop(0, N // 16)
    def _zero(i):
        hist_v[pl.ds(i * 16, 16)] = z16

    ones16 = jnp.ones((16,), jnp.float32)

    def _part(i, offs):
        off_lo, off_hi = offs
        s16 = src_v[i]
        d16 = dst_v[i]
        plsc.addupdate_scatter(hist_v, [d16], ones16)
        e16 = jnp.bitwise_or(jnp.left_shift(s16, 14), d16)
        m = d16 < NH
        plsc.store_compressed(elo_v.at[pl.ds(off_lo, 16)], e16, mask=m)
        mh = jnp.logical_not(m)
        plsc.store_compressed(ehi_v.at[pl.ds(off_hi, 16)], e16, mask=mh)
        nlo = jnp.max(plsc.all_reduce_population_count(m))
        return off_lo + nlo, off_hi + (16 - nlo)

    def _block(b, offs):
        pltpu.sync_copy(src_hbm.at[wid, b], src_v)
        pltpu.sync_copy(dst_hbm.at[wid, b], dst_v)
        return lax.fori_loop(0, 125, _part, offs)

    off_lo, off_hi = lax.fori_loop(0, 5, _block, (0, 0))

    pad_lo = jnp.full((16,), NH, jnp.int32)
    pad_hi = jnp.full((16,), 2 * NH, jnp.int32)
    for k in range(CH // 16):
        elo_v[pl.ds(off_lo + k * 16, 16)] = pad_lo
        ehi_v[pl.ds(off_hi + k * 16, 16)] = pad_hi

    cnt_v[0, pl.ds(0, 16)] = jnp.full((16,), off_lo, jnp.int32)
    cnt_v[1, pl.ds(0, 16)] = jnp.full((16,), off_hi, jnp.int32)

    pltpu.sync_copy(hist_v, deg_hbm.at[wid])
    pltpu.sync_copy(elo_v, elo_hbm.at[wid])
    pltpu.sync_copy(ehi_v, ehi_hbm.at[wid])
    pltpu.sync_copy(cnt_v, cnt_hbm.at[wid])


@functools.partial(
    pl.kernel,
    out_type=jax.ShapeDtypeStruct((2, N, D), jnp.float32),
    mesh=_mesh,
    scratch_types=[
        pltpu.VMEM((LCH, CH), jnp.int32),
        pltpu.VMEM((LCH, CH), jnp.int32),
        pltpu.VMEM((2, 16), jnp.int32),
        pltpu.VMEM((8, CH), jnp.int32),
        pltpu.VMEM((8, CH), jnp.int32),
        pltpu.VMEM((CH, D), jnp.float32),
        pltpu.VMEM((CH, D), jnp.float32),
        pltpu.VMEM((200, D), jnp.float32),
        pltpu.VMEM_SHARED((ACC_ROWS, D), jnp.float32),
        pltpu.SemaphoreType.DMA,
        pltpu.SemaphoreType.DMA,
    ],
    compiler_params=pltpu.CompilerParams(needs_layout_passes=False),
)
def _msg_kernel(hn_hbm, elo_hbm, ehi_hbm, cnt_hbm,
                out_hbm, elo_v, ehi_v, cnt_v, srcb, idxb,
                rows0, rows1, zbuf, acc_sh, sem0, sem1):
    sid = lax.axis_index("s")
    cid = lax.axis_index("c")
    wid = sid * 2 + cid

    z16 = jnp.zeros((16,), jnp.float32)

    @pl.loop(0, 200)
    def _zrow(i):
        @pl.loop(0, D // 16)
        def _zcol(c):
            zbuf[i, pl.ds(c * 16, 16)] = z16

    pltpu.sync_copy(elo_hbm.at[wid], elo_v)
    pltpu.sync_copy(ehi_hbm.at[wid], ehi_v)
    pltpu.sync_copy(cnt_hbm.at[wid], cnt_v)

    for p, ev in ((0, elo_v), (1, ehi_v)):
        base = p * NH

        @pl.when(sid < 5)
        def _zacc():
            @pl.loop(0, 5)
            def _zk(k):
                pltpu.sync_copy(
                    zbuf, acc_sh.at[pl.ds(sid * 1000 + k * 200, 200), :])

        @pl.when(sid == 5)
        def _zdump():
            pltpu.sync_copy(zbuf.at[pl.ds(0, 8), :],
                            acc_sh.at[pl.ds(NH, 8), :])

        plsc.subcore_barrier()

        cnt = jnp.max(cnt_v[p])
        nch = jnp.maximum((cnt + CH - 1) // CH, 1)

        def _start(j, buf, sem_, row):
            e_row = ev.at[j]
            for k in range(CH // 16):
                e16 = e_row[pl.ds(k * 16, 16)]
                srcb[row, pl.ds(k * 16, 16)] = jnp.right_shift(e16, 14)
                idxb[row, pl.ds(k * 16, 16)] = (
                    jnp.bitwise_and(e16, 16383) - base)
            pltpu.async_copy(hn_hbm.at[srcb.at[row]], buf, sem_)

        def _finish(buf, sem_, row):
            pltpu.make_async_copy(hn_hbm.at[pl.ds(0, CH), :], buf, sem_).wait()
            pltpu.sync_copy(buf, acc_sh.at[idxb.at[row]], add=True)

        _start(0, rows0, sem0, 0)

        def _chunk(j, carry):
            even = lax.rem(j, 2) == 0

            @pl.when(even)
            def _even():
                @pl.when(j + 1 < nch)
                def _pre():
                    _start(j + 1, rows1, sem1, 1)
                _finish(rows0, sem0, 0)

            @pl.when(jnp.logical_not(even))
            def _odd():
                @pl.when(j + 1 < nch)
                def _pre():
                    _start(j + 1, rows0, sem0, 0)
                _finish(rows1, sem1, 1)

            return carry

        lax.fori_loop(0, nch, _chunk, 0)

        plsc.subcore_barrier()

        @pl.when(sid < 5)
        def _writeout():
            pltpu.sync_copy(
                acc_sh.at[pl.ds(sid * 1000, 1000), :],
                out_hbm.at[cid, pl.ds(base + sid * 1000, 1000), :],
            )

        plsc.subcore_barrier()


def _tc_a_body(x_ref, w_ref, dg_ref, out_ref):
    deg = jnp.sum(dg_ref[:, :], axis=1) + 1.0
    dinv = lax.rsqrt(deg)
    h = jnp.dot(x_ref[:, :], w_ref[:, :], preferred_element_type=jnp.float32)
    out_ref[:, :] = h * dinv[:, None]


def _tc_a(x, w, dg, rows):
    grid = x.shape[0] // rows
    return pl.pallas_call(
        _tc_a_body,
        grid=(grid,),
        in_specs=[
            pl.BlockSpec((rows, D), lambda i: (i, 0)),
            pl.BlockSpec((D, D), lambda i: (0, 0)),
            pl.BlockSpec((rows, NW), lambda i: (i, 0)),
        ],
        out_specs=pl.BlockSpec((rows, D), lambda i: (i, 0)),
        out_shape=jax.ShapeDtypeStruct((x.shape[0], D), jnp.float32),
    )(x, w, dg)


def _tc_c_body(a0_ref, a1_ref, hn_ref, dg_ref, b_ref, w_ref, out_ref):
    deg = jnp.sum(dg_ref[:, :], axis=1) + 1.0
    dinv = lax.rsqrt(deg)
    t = (a0_ref[:, :] + a1_ref[:, :] + hn_ref[:, :]) * dinv[:, None]
    t = jnp.maximum(t + b_ref[:, :], 0.0)
    h = jnp.dot(t, w_ref[:, :], preferred_element_type=jnp.float32)
    out_ref[:, :] = h * dinv[:, None]


def _tc_c(a0, a1, hn, dg, b, w, rows):
    grid = hn.shape[0] // rows
    return pl.pallas_call(
        _tc_c_body,
        grid=(grid,),
        in_specs=[
            pl.BlockSpec((rows, D), lambda i: (i, 0)),
            pl.BlockSpec((rows, D), lambda i: (i, 0)),
            pl.BlockSpec((rows, D), lambda i: (i, 0)),
            pl.BlockSpec((rows, NW), lambda i: (i, 0)),
            pl.BlockSpec((1, D), lambda i: (0, 0)),
            pl.BlockSpec((D, D), lambda i: (0, 0)),
        ],
        out_specs=pl.BlockSpec((rows, D), lambda i: (i, 0)),
        out_shape=jax.ShapeDtypeStruct((hn.shape[0], D), jnp.float32),
    )(a0, a1, hn, dg, b, w)


def _tc_e_body(a0_ref, a1_ref, hn_ref, dg_ref, b_ref, batch_ref,
               wfc_ref, bfc_ref, out_ref, gsum, gcnt):
    i = pl.program_id(0)

    @pl.when(i == 0)
    def _init():
        gsum[:, :] = jnp.zeros_like(gsum)
        gcnt[:, :] = jnp.zeros_like(gcnt)

    deg = jnp.sum(dg_ref[:, :], axis=1) + 1.0
    dinv = lax.rsqrt(deg)
    t = (a0_ref[:, :] + a1_ref[:, :] + hn_ref[:, :]) * dinv[:, None]
    h3 = jnp.maximum(t + b_ref[:, :], 0.0)

    rows = h3.shape[0]
    gids = lax.broadcasted_iota(jnp.int32, (G, rows), 0)
    mask = (batch_ref[:, 0][None, :] == gids).astype(jnp.float32)
    gsum[:, :] += jnp.dot(mask, h3, preferred_element_type=jnp.float32)
    cnt = jnp.sum(mask, axis=1, keepdims=True)
    gcnt[:, :] += jnp.broadcast_to(cnt, (G, D))

    @pl.when(i == pl.num_programs(0) - 1)
    def _fin():
        g = gsum[:, :] / jnp.maximum(gcnt[:, :], 1.0)
        out_ref[:, :] = (
            jnp.dot(g, wfc_ref[:, :], preferred_element_type=jnp.float32)
            + bfc_ref[:, :])


def _tc_e(a0, a1, hn, dg, b, batch2d, wfc, bfc, rows):
    grid = hn.shape[0] // rows
    return pl.pallas_call(
        _tc_e_body,
        grid=(grid,),
        in_specs=[
            pl.BlockSpec((rows, D), lambda i: (i, 0)),
            pl.BlockSpec((rows, D), lambda i: (i, 0)),
            pl.BlockSpec((rows, D), lambda i: (i, 0)),
            pl.BlockSpec((rows, NW), lambda i: (i, 0)),
            pl.BlockSpec((1, D), lambda i: (0, 0)),
            pl.BlockSpec((rows, 1), lambda i: (i, 0)),
            pl.BlockSpec((D, D), lambda i: (0, 0)),
            pl.BlockSpec((1, D), lambda i: (0, 0)),
        ],
        out_specs=pl.BlockSpec((G, D), lambda i: (0, 0)),
        out_shape=jax.ShapeDtypeStruct((G, D), jnp.float32),
        scratch_shapes=[
            pltpu.VMEM((G, D), jnp.float32),
            pltpu.VMEM((G, D), jnp.float32),
        ],
    )(a0, a1, hn, dg, b, batch2d, wfc, bfc)


def kernel(x, edge_index, batch, W1, b1, W2, b2, Wfc, bfc):
    src = edge_index[0]
    dst = edge_index[1]
    src_r = src.reshape(NW, 5, 125, 16)
    dst_r = dst.reshape(NW, 5, 125, 16)

    deg_p, elo, ehi, cnts = _deg_kernel(src_r, dst_r)
    dg = deg_p.T
    elo = elo.reshape(NW, LCH, CH)
    ehi = ehi.reshape(NW, LCH, CH)

    rows = 1000
    hn1 = _tc_a(x, W1, dg, rows)
    acc1 = _msg_kernel(hn1, elo, ehi, cnts)
    hn2 = _tc_c(acc1[0], acc1[1], hn1, dg,
                b1.reshape(1, D), W2, rows)
    acc2 = _msg_kernel(hn2, elo, ehi, cnts)
    out = _tc_e(acc2[0], acc2[1], hn2, dg,
                b2.reshape(1, D), batch.reshape(N, 1),
                Wfc, bfc.reshape(1, D), rows)
    return out
